# Initial kernel scaffold; baseline (speedup 1.0000x reference)
import jax
import jax.numpy as jnp
from jax.experimental import pallas as pl


def kernel(x, edge_index, W1, b1, W2, b2):
    raise NotImplementedError("write your pallas kernel here")


# speedup vs baseline: 54.2543x; 54.2543x over previous
import functools

import jax
import jax.numpy as jnp
from jax import lax
from jax.experimental import pallas as pl
from jax.experimental.pallas import tpu as pltpu
from jax.experimental.pallas import tpu_sc as plsc

N_NODES = 10000
N_PAD = 10240
D_IN = 128
D_HID = 16
D_OUT = 2
N_EDGES = 320000
CHUNK = 125
NW = 32
CPW = 80
EPW = CPW * CHUNK
NBUF = 5
NGRP = CPW // NBUF
RPT = N_NODES // 16

_f32 = jnp.float32

_MESH = plsc.VectorSubcoreMesh(core_axis_name="c", subcore_axis_name="s")
_SC_PARAMS = pltpu.CompilerParams(
    needs_layout_passes=False, use_tc_tiling_on_sc=False)


@functools.partial(
    pl.kernel,
    out_type=jax.ShapeDtypeStruct((NW, N_PAD // 16, 16), _f32),
    mesh=_MESH,
    compiler_params=_SC_PARAMS,
    scratch_types=[
        pltpu.VMEM((CPW, CHUNK), jnp.int32),
        pltpu.VMEM((N_PAD // 16, 16), _f32),
    ],
)
def _deg_kernel(dst_hbm, out_hbm, didx, deg):
    c = lax.axis_index("c")
    s = lax.axis_index("s")
    w = s * 2 + c

    zeros16 = jnp.zeros((16,), _f32)
    ones16 = jnp.ones((16,), _f32)

    def zbody(i, carry):
        deg[i, :] = zeros16
        return carry

    lax.fori_loop(0, N_PAD // 16, zbody, 0)

    pltpu.sync_copy(dst_hbm.at[w], didx)

    def body(r, carry):
        for k in range(CHUNK // 16):
            idx = didx[r, pl.ds(k * 16, 16)]
            plsc.addupdate_scatter(
                deg,
                [jnp.right_shift(idx, 4), jnp.bitwise_and(idx, 15)],
                ones16,
            )
        tail = didx[r, pl.ds(CHUNK - 16, 16)]
        lanes = lax.iota(jnp.int32, 16)
        plsc.addupdate_scatter(
            deg,
            [jnp.right_shift(tail, 4), jnp.bitwise_and(tail, 15)],
            ones16,
            mask=lanes >= (16 - (CHUNK % 16)),
        )
        return carry

    lax.fori_loop(0, CPW, body, 0)

    pltpu.sync_copy(deg, out_hbm.at[w])


@functools.partial(
    pl.kernel,
    out_type=jax.ShapeDtypeStruct((2, 16, RPT, D_HID), _f32),
    mesh=_MESH,
    compiler_params=_SC_PARAMS,
    scratch_types=[
        pltpu.VMEM((CPW, CHUNK), jnp.int32),
        pltpu.VMEM((CPW, CHUNK), jnp.int32),
        pltpu.VMEM((NBUF, CHUNK, D_HID), _f32),
        pltpu.VMEM_SHARED((N_NODES, D_HID), _f32),
        pltpu.SemaphoreType.DMA,
        pltpu.SemaphoreType.DMA,
        pltpu.SemaphoreType.DMA,
        pltpu.SemaphoreType.DMA,
        pltpu.SemaphoreType.DMA,
    ],
)
def _scatter_kernel(g_hbm, src_hbm, dst_hbm, zero_hbm, out_hbm,
                    sidx, didx, rows, acc, s0, s1, s2, s3, s4):
    sems = [s0, s1, s2, s3, s4]
    c = lax.axis_index("c")
    s = lax.axis_index("s")
    w = s * 2 + c

    pltpu.sync_copy(src_hbm.at[w], sidx)
    pltpu.sync_copy(dst_hbm.at[w], didx)

    pltpu.sync_copy(zero_hbm.at[s], acc.at[pl.ds(s * RPT, RPT)])
    plsc.subcore_barrier()

    for b in range(NBUF):
        pltpu.make_async_copy(g_hbm.at[sidx.at[b]], rows.at[b], sems[b]).start()

    def grp(gi, carry):
        for b in range(NBUF):
            j = gi * NBUF + b
            pltpu.make_async_copy(g_hbm.at[sidx.at[j]], rows.at[b],
                                  sems[b]).wait()
            pltpu.sync_copy(rows.at[b], acc.at[didx.at[j]], add=True)

            @pl.when(gi < NGRP - 1)
            def _():
                pltpu.make_async_copy(g_hbm.at[sidx.at[j + NBUF]], rows.at[b],
                                      sems[b]).start()
        return carry

    lax.fori_loop(0, NGRP, grp, 0)
    plsc.subcore_barrier()

    pltpu.sync_copy(acc.at[pl.ds(s * RPT, RPT)], out_hbm.at[c, s])


def _b1_body(p_ref, o_ref):
    o_ref[...] = lax.rsqrt(jnp.sum(p_ref[...], axis=0) + 1.0)


def _b2_body(x_ref, w_ref, d_ref, o_ref):
    h = jnp.dot(x_ref[...], w_ref[...], preferred_element_type=_f32)
    o_ref[...] = h * d_ref[...]


def _d_body(s_ref, g_ref, d_ref, b_ref, o_ref):
    t = s_ref[0] + s_ref[1] + g_ref[...]
    a = d_ref[...] * t + b_ref[...]
    o_ref[...] = d_ref[...] * jnp.maximum(a, 0.0)


def _f_body(s_ref, g_ref, d_ref, w_ref, b_ref, o_ref):
    z = d_ref[...] * (s_ref[0] + s_ref[1] + g_ref[...])
    logits = jnp.dot(z, w_ref[...], preferred_element_type=_f32) + b_ref[...]
    m = jnp.max(logits, axis=1, keepdims=True)
    sh = logits - m
    o_ref[...] = sh - jnp.log(jnp.sum(jnp.exp(sh), axis=1, keepdims=True))


def kernel(x, edge_index, W1, b1, W2, b2):
    src3d = edge_index[0].reshape(NW, CPW, CHUNK)
    dst3d = edge_index[1].reshape(NW, CPW, CHUNK)
    zeros = jnp.zeros((16, RPT, D_HID), _f32)

    deg_parts = _deg_kernel(dst3d)

    dis80 = pl.pallas_call(
        _b1_body,
        out_shape=jax.ShapeDtypeStruct((N_PAD // 128, 128), _f32),
    )(deg_parts.reshape(NW, N_PAD // 128, 128))
    dis_col = dis80.reshape(N_PAD)[:N_NODES].reshape(N_NODES, 1)

    g1 = pl.pallas_call(
        _b2_body,
        out_shape=jax.ShapeDtypeStruct((N_NODES, D_HID), _f32),
    )(x, W1, dis_col)

    s1 = _scatter_kernel(g1, src3d, dst3d, zeros)
    s1 = s1.reshape(2, N_NODES, D_HID)

    g2 = pl.pallas_call(
        _d_body,
        out_shape=jax.ShapeDtypeStruct((N_NODES, D_HID), _f32),
    )(s1, g1, dis_col, b1.reshape(1, D_HID))

    s2 = _scatter_kernel(g2, src3d, dst3d, zeros)
    s2 = s2.reshape(2, N_NODES, D_HID)

    out = pl.pallas_call(
        _f_body,
        out_shape=jax.ShapeDtypeStruct((N_NODES, D_OUT), _f32),
    )(s2, g2, dis_col, W2, b2.reshape(1, D_OUT))
    return out


# speedup vs baseline: 69.2685x; 1.2767x over previous
import functools

import jax
import jax.numpy as jnp
from jax import lax
from jax.experimental import pallas as pl
from jax.experimental.pallas import tpu as pltpu
from jax.experimental.pallas import tpu_sc as plsc

N_NODES = 10000
N_PAD = 10240
D_IN = 128
D_HID = 16
D_OUT = 2
N_EDGES = 320000
CHUNK = 125
NW = 32
CPW = 80
NBUF = 5
NGRP = CPW // NBUF
NPT = N_PAD // 16
DRT = NPT // 16

_f32 = jnp.float32

_MESH = plsc.VectorSubcoreMesh(core_axis_name="c", subcore_axis_name="s")
_SC_PARAMS = pltpu.CompilerParams(
    needs_layout_passes=False, use_tc_tiling_on_sc=False)


def _newton_rsqrt(d):
    h = d * 0.5
    i = plsc.bitcast(d, jnp.int32)
    i = 0x5F3759DF - jnp.right_shift(i, 1)
    y = plsc.bitcast(i, _f32)
    for _ in range(3):
        y = y * (1.5 - h * y * y)
    return y


@functools.partial(
    pl.kernel,
    out_type=jax.ShapeDtypeStruct((NW, NPT, 16), _f32),
    mesh=_MESH,
    compiler_params=_SC_PARAMS,
    scratch_types=[
        pltpu.VMEM((CPW, CHUNK), jnp.int32),
        pltpu.VMEM((NPT, 16), _f32),
    ],
)
def _deg_kernel(dst_hbm, out_hbm, didx, deg):
    c = lax.axis_index("c")
    s = lax.axis_index("s")
    w = s * 2 + c

    zeros16 = jnp.zeros((16,), _f32)
    ones16 = jnp.ones((16,), _f32)

    def zbody(i, carry):
        deg[i, :] = zeros16
        return carry

    lax.fori_loop(0, NPT, zbody, 0)

    pltpu.sync_copy(dst_hbm.at[w], didx)

    def body(r, carry):
        for k in range(CHUNK // 16):
            idx = didx[r, pl.ds(k * 16, 16)]
            plsc.addupdate_scatter(
                deg,
                [jnp.right_shift(idx, 4), jnp.bitwise_and(idx, 15)],
                ones16,
            )
        tail = didx[r, pl.ds(CHUNK - 16, 16)]
        lanes = lax.iota(jnp.int32, 16)
        plsc.addupdate_scatter(
            deg,
            [jnp.right_shift(tail, 4), jnp.bitwise_and(tail, 15)],
            ones16,
            mask=lanes >= (16 - (CHUNK % 16)),
        )
        return carry

    lax.fori_loop(0, CPW, body, 0)

    pltpu.sync_copy(deg, out_hbm.at[w])


def _edge_pipeline(tbl_sh, acc, sidx, didx, rows, gsem, ssem):
    for b in range(NBUF):
        pltpu.make_async_copy(tbl_sh.at[sidx.at[b]], rows.at[0, b],
                              gsem.at[0, b]).start()

    def grp2(gi2, carry):
        for ri in range(2):
            gi = gi2 * 2 + ri
            ro = 1 - ri
            for b in range(NBUF):
                j = gi * NBUF + b
                pltpu.make_async_copy(tbl_sh.at[sidx.at[j]], rows.at[ri, b],
                                      gsem.at[ri, b]).wait()
                pltpu.async_copy(rows.at[ri, b], acc.at[didx.at[j]],
                                 ssem.at[ri, b], add=True)

                @pl.when(gi < NGRP - 1)
                def _():
                    @pl.when(gi > 0)
                    def _():
                        pltpu.make_async_copy(
                            rows.at[ro, b], acc.at[didx.at[j]],
                            ssem.at[ro, b]).wait()

                    pltpu.make_async_copy(
                        tbl_sh.at[sidx.at[j + NBUF]], rows.at[ro, b],
                        gsem.at[ro, b]).start()
        return carry

    lax.fori_loop(0, NGRP // 2, grp2, 0)
    for ri in range(2):
        for b in range(NBUF):
            pltpu.make_async_copy(rows.at[ri, b], acc.at[didx.at[b]],
                                  ssem.at[ri, b]).wait()


_EDGE_SCRATCH = [
    pltpu.VMEM((CPW, CHUNK), jnp.int32),
    pltpu.VMEM((CPW, CHUNK), jnp.int32),
    pltpu.VMEM((2, NBUF, CHUNK, D_HID), _f32),
    pltpu.VMEM_SHARED((N_PAD, D_HID), _f32),
    pltpu.VMEM_SHARED((N_PAD, D_HID), _f32),
    pltpu.SemaphoreType.DMA((2, NBUF)),
    pltpu.SemaphoreType.DMA((2, NBUF)),
    pltpu.SemaphoreType.DMA,
]


@functools.partial(
    pl.kernel,
    out_type=(
        jax.ShapeDtypeStruct((2, 16, NPT, D_HID), _f32),
        jax.ShapeDtypeStruct((16, NPT, D_HID), _f32),
        jax.ShapeDtypeStruct((16, NPT, D_HID), _f32),
    ),
    mesh=_MESH,
    compiler_params=_SC_PARAMS,
    scratch_types=_EDGE_SCRATCH + [
        pltpu.VMEM((NW, DRT, 16), _f32),
        pltpu.VMEM((NPT, D_HID), _f32),
        pltpu.VMEM((NPT, D_HID), _f32),
        pltpu.VMEM((NPT, D_HID), _f32),
    ],
)
def _layer1_kernel(h1_hbm, deg_hbm, src_hbm, dst_hbm, zero_hbm,
                   out_hbm, g1_hbm, dis_hbm,
                   sidx, didx, rows, tbl, acc, gsem, ssem, stsem,
                   dpb, h1b, g1b, db):
    c = lax.axis_index("c")
    s = lax.axis_index("s")
    w = s * 2 + c

    pltpu.sync_copy(src_hbm.at[w], sidx)
    pltpu.sync_copy(dst_hbm.at[w], didx)
    pltpu.sync_copy(zero_hbm.at[s], acc.at[pl.ds(s * NPT, NPT)])

    for k in range(NW):
        pltpu.make_async_copy(deg_hbm.at[k, pl.ds(s * DRT, DRT)],
                              dpb.at[k], stsem).start()
    pltpu.make_async_copy(h1_hbm.at[s], h1b, stsem).start()
    for k in range(NW):
        pltpu.make_async_copy(deg_hbm.at[k, pl.ds(s * DRT, DRT)],
                              dpb.at[k], stsem).wait()
    pltpu.make_async_copy(h1_hbm.at[s], h1b, stsem).wait()

    def drow(r, carry):
        cnt = dpb[0, r, :]
        for k in range(1, NW):
            cnt = cnt + dpb[k, r, :]
        y = _newton_rsqrt(cnt + 1.0)
        for k in range(16):
            dv = y[k]
            rr = r * 16 + k
            db[rr, :] = jnp.zeros((16,), _f32) + dv
            g1b[rr, :] = h1b[rr, :] * dv
        return carry

    lax.fori_loop(0, DRT, drow, 0)

    pltpu.sync_copy(g1b, tbl.at[pl.ds(s * NPT, NPT)])

    @pl.when(c == 0)
    def _():
        pltpu.sync_copy(g1b, g1_hbm.at[s])
        pltpu.sync_copy(db, dis_hbm.at[s])

    plsc.subcore_barrier()

    _edge_pipeline(tbl, acc, sidx, didx, rows, gsem, ssem)

    plsc.subcore_barrier()
    pltpu.sync_copy(acc.at[pl.ds(s * NPT, NPT)], out_hbm.at[c, s])


@functools.partial(
    pl.kernel,
    out_type=(
        jax.ShapeDtypeStruct((2, 16, NPT, D_HID), _f32),
        jax.ShapeDtypeStruct((16, NPT, D_HID), _f32),
    ),
    mesh=_MESH,
    compiler_params=_SC_PARAMS,
    scratch_types=_EDGE_SCRATCH + [
        pltpu.VMEM((NPT, D_HID), _f32),
        pltpu.VMEM((NPT, D_HID), _f32),
        pltpu.VMEM((NPT, D_HID), _f32),
        pltpu.VMEM((NPT, D_HID), _f32),
        pltpu.VMEM((NPT, D_HID), _f32),
        pltpu.VMEM((16,), _f32),
    ],
)
def _layer2_kernel(s1_hbm, g1_hbm, dis_hbm, b1_hbm, src_hbm, dst_hbm,
                   zero_hbm, out_hbm, g2_hbm,
                   sidx, didx, rows, tbl, acc, gsem, ssem, stsem,
                   p0b, p1b, g1b, db, g2b, b1b):
    c = lax.axis_index("c")
    s = lax.axis_index("s")
    w = s * 2 + c

    pltpu.sync_copy(src_hbm.at[w], sidx)
    pltpu.sync_copy(dst_hbm.at[w], didx)
    pltpu.sync_copy(zero_hbm.at[s], acc.at[pl.ds(s * NPT, NPT)])

    pltpu.make_async_copy(s1_hbm.at[0, s], p0b, stsem).start()
    pltpu.make_async_copy(s1_hbm.at[1, s], p1b, stsem).start()
    pltpu.make_async_copy(g1_hbm.at[s], g1b, stsem).start()
    pltpu.make_async_copy(dis_hbm.at[s], db, stsem).start()
    pltpu.make_async_copy(b1_hbm, b1b, stsem).start()
    pltpu.make_async_copy(s1_hbm.at[0, s], p0b, stsem).wait()
    pltpu.make_async_copy(s1_hbm.at[1, s], p1b, stsem).wait()
    pltpu.make_async_copy(g1_hbm.at[s], g1b, stsem).wait()
    pltpu.make_async_copy(dis_hbm.at[s], db, stsem).wait()
    pltpu.make_async_copy(b1_hbm, b1b, stsem).wait()

    b1v = b1b[...]

    def grow(rr, carry):
        d = db[rr, :]
        a = d * (p0b[rr, :] + p1b[rr, :] + g1b[rr, :]) + b1v
        g2b[rr, :] = d * jnp.maximum(a, 0.0)
        return carry

    lax.fori_loop(0, NPT, grow, 0)

    pltpu.sync_copy(g2b, tbl.at[pl.ds(s * NPT, NPT)])

    @pl.when(c == 0)
    def _():
        pltpu.sync_copy(g2b, g2_hbm.at[s])

    plsc.subcore_barrier()

    _edge_pipeline(tbl, acc, sidx, didx, rows, gsem, ssem)

    plsc.subcore_barrier()
    pltpu.sync_copy(acc.at[pl.ds(s * NPT, NPT)], out_hbm.at[c, s])


def _h1_body(x_ref, w_ref, o_ref):
    o_ref[...] = jnp.dot(x_ref[...], w_ref[...],
                         preferred_element_type=_f32)[None]


def _f_body(s_ref, g_ref, d_ref, w_ref, b_ref, o_ref):
    z3 = d_ref[...] * (s_ref[0] + s_ref[1] + g_ref[...])
    z = z3.reshape(N_PAD, D_HID)
    logits = jnp.dot(z, w_ref[...], preferred_element_type=_f32) + b_ref[...]
    m = jnp.max(logits, axis=1, keepdims=True)
    sh = logits - m
    lsm = sh - jnp.log(jnp.sum(jnp.exp(sh), axis=1, keepdims=True))
    o_ref[...] = lsm[:N_NODES, :]


def kernel(x, edge_index, W1, b1, W2, b2):
    src3d = edge_index[0].reshape(NW, CPW, CHUNK)
    dst3d = edge_index[1].reshape(NW, CPW, CHUNK)
    zeros = jnp.zeros((16, NPT, D_HID), _f32)

    h1 = pl.pallas_call(
        _h1_body,
        grid=(16,),
        in_specs=[
            pl.BlockSpec((NPT, D_IN), lambda i: (i, 0)),
            pl.BlockSpec((D_IN, D_HID), lambda i: (0, 0)),
        ],
        out_specs=pl.BlockSpec((1, NPT, D_HID), lambda i: (i, 0, 0)),
        out_shape=jax.ShapeDtypeStruct((16, NPT, D_HID), _f32),
    )(x, W1)

    deg_parts = _deg_kernel(dst3d)

    s1, g1, dis = _layer1_kernel(h1, deg_parts, src3d, dst3d, zeros)

    s2, g2 = _layer2_kernel(s1, g1, dis, b1, src3d, dst3d, zeros)

    out = pl.pallas_call(
        _f_body,
        out_shape=jax.ShapeDtypeStruct((N_NODES, D_OUT), _f32),
    )(s2, g2, dis, W2, b2.reshape(1, D_OUT))
    return out


# speedup vs baseline: 93.6084x; 1.3514x over previous
import functools

import jax
import jax.numpy as jnp
from jax import lax
from jax.experimental import pallas as pl
from jax.experimental.pallas import tpu as pltpu
from jax.experimental.pallas import tpu_sc as plsc

N_NODES = 10000
N_PAD = 10240
D_IN = 128
D_HID = 16
D_OUT = 2
N_EDGES = 320000
CHUNK = 80
NW = 32
CPW = 125
NBUF = 5
NGRP = CPW // NBUF
NPT = N_PAD // 16
DRT = NPT // 16

_f32 = jnp.float32

_MESH = plsc.VectorSubcoreMesh(core_axis_name="c", subcore_axis_name="s")
_SC_PARAMS = pltpu.CompilerParams(
    needs_layout_passes=False, use_tc_tiling_on_sc=False)


def _newton_rsqrt(d):
    h = d * 0.5
    i = plsc.bitcast(d, jnp.int32)
    i = 0x5F3759DF - jnp.right_shift(i, 1)
    y = plsc.bitcast(i, _f32)
    for _ in range(3):
        y = y * (1.5 - h * y * y)
    return y


@functools.partial(
    pl.kernel,
    out_type=jax.ShapeDtypeStruct((NW, NPT, 16), _f32),
    mesh=_MESH,
    compiler_params=_SC_PARAMS,
    scratch_types=[
        pltpu.VMEM((CPW, CHUNK), jnp.int32),
        pltpu.VMEM((NPT, 16), _f32),
    ],
)
def _deg_kernel(edges_hbm, out_hbm, didx, deg):
    c = lax.axis_index("c")
    s = lax.axis_index("s")
    w = s * 2 + c

    zeros16 = jnp.zeros((16,), _f32)
    ones16 = jnp.ones((16,), _f32)

    def zbody(i, carry):
        deg[i, :] = zeros16
        return carry

    lax.fori_loop(0, NPT, zbody, 0)

    pltpu.sync_copy(edges_hbm.at[1, w], didx)

    def body(r, carry):
        for k in range(CHUNK // 16):
            idx = didx[r, pl.ds(k * 16, 16)]
            plsc.addupdate_scatter(
                deg,
                [jnp.right_shift(idx, 4), jnp.bitwise_and(idx, 15)],
                ones16,
            )
        return carry

    lax.fori_loop(0, CPW, body, 0)

    pltpu.sync_copy(deg, out_hbm.at[w])


def _edge_pipeline(tbl_sh, acc, sidx, didx, rows, gsem, ssem):
    for b in range(NBUF):
        pltpu.make_async_copy(tbl_sh.at[sidx.at[b]], rows.at[0, b],
                              gsem.at[0, b]).start()

    def one_group(gi, ri):
        ro = 1 - ri
        for b in range(NBUF):
            j = gi * NBUF + b
            pltpu.make_async_copy(tbl_sh.at[sidx.at[j]], rows.at[ri, b],
                                  gsem.at[ri, b]).wait()
            pltpu.async_copy(rows.at[ri, b], acc.at[didx.at[j]],
                             ssem.at[ri, b], add=True)

            @pl.when(gi < NGRP - 1)
            def _():
                @pl.when(gi > 0)
                def _():
                    pltpu.make_async_copy(
                        rows.at[ro, b], acc.at[didx.at[j]],
                        ssem.at[ro, b]).wait()

                pltpu.make_async_copy(
                    tbl_sh.at[sidx.at[j + NBUF]], rows.at[ro, b],
                    gsem.at[ro, b]).start()

    def grp2(gi2, carry):
        for ri in range(2):
            one_group(gi2 * 2 + ri, ri)
        return carry

    lax.fori_loop(0, (NGRP - 1) // 2, grp2, 0)
    one_group(NGRP - 1, (NGRP - 1) % 2)
    for ri in range(2):
        for b in range(NBUF):
            pltpu.make_async_copy(rows.at[ri, b], acc.at[didx.at[b]],
                                  ssem.at[ri, b]).wait()


_EDGE_SCRATCH = [
    pltpu.VMEM((CPW, CHUNK), jnp.int32),
    pltpu.VMEM((CPW, CHUNK), jnp.int32),
    pltpu.VMEM((2, NBUF, CHUNK, D_HID), _f32),
    pltpu.VMEM_SHARED((N_PAD, D_HID), _f32),
    pltpu.VMEM_SHARED((N_PAD, D_HID), _f32),
    pltpu.SemaphoreType.DMA((2, NBUF)),
    pltpu.SemaphoreType.DMA((2, NBUF)),
    pltpu.SemaphoreType.DMA,
]


@functools.partial(
    pl.kernel,
    out_type=(
        jax.ShapeDtypeStruct((2, 16, NPT, D_HID), _f32),
        jax.ShapeDtypeStruct((16, NPT, D_HID), _f32),
        jax.ShapeDtypeStruct((16, NPT, D_HID), _f32),
    ),
    mesh=_MESH,
    compiler_params=_SC_PARAMS,
    scratch_types=_EDGE_SCRATCH + [
        pltpu.VMEM((NW, DRT, 16), _f32),
        pltpu.VMEM((NPT, D_HID), _f32),
        pltpu.VMEM((NPT, D_HID), _f32),
        pltpu.VMEM((NPT, D_HID), _f32),
    ],
)
def _layer1_kernel(h1_hbm, deg_hbm, edges_hbm, zero_hbm,
                   out_hbm, g1_hbm, dis_hbm,
                   sidx, didx, rows, tbl, acc, gsem, ssem, stsem,
                   dpb, h1b, g1b, db):
    c = lax.axis_index("c")
    s = lax.axis_index("s")
    w = s * 2 + c

    pltpu.sync_copy(edges_hbm.at[0, w], sidx)
    pltpu.sync_copy(edges_hbm.at[1, w], didx)
    pltpu.sync_copy(zero_hbm.at[s], acc.at[pl.ds(s * NPT, NPT)])

    for k in range(NW):
        pltpu.make_async_copy(deg_hbm.at[k, pl.ds(s * DRT, DRT)],
                              dpb.at[k], stsem).start()
    pltpu.make_async_copy(h1_hbm.at[s], h1b, stsem).start()
    for k in range(NW):
        pltpu.make_async_copy(deg_hbm.at[k, pl.ds(s * DRT, DRT)],
                              dpb.at[k], stsem).wait()
    pltpu.make_async_copy(h1_hbm.at[s], h1b, stsem).wait()

    def drow(r, carry):
        cnt = dpb[0, r, :]
        for k in range(1, NW):
            cnt = cnt + dpb[k, r, :]
        y = _newton_rsqrt(cnt + 1.0)
        for k in range(16):
            dv = y[k]
            rr = r * 16 + k
            db[rr, :] = jnp.zeros((16,), _f32) + dv
            g1b[rr, :] = h1b[rr, :] * dv
        return carry

    lax.fori_loop(0, DRT, drow, 0)

    pltpu.sync_copy(g1b, tbl.at[pl.ds(s * NPT, NPT)])

    @pl.when(c == 0)
    def _():
        pltpu.sync_copy(g1b, g1_hbm.at[s])
        pltpu.sync_copy(db, dis_hbm.at[s])

    plsc.subcore_barrier()

    _edge_pipeline(tbl, acc, sidx, didx, rows, gsem, ssem)

    plsc.subcore_barrier()
    pltpu.sync_copy(acc.at[pl.ds(s * NPT, NPT)], out_hbm.at[c, s])


@functools.partial(
    pl.kernel,
    out_type=jax.ShapeDtypeStruct((2, D_OUT, N_PAD), _f32),
    mesh=_MESH,
    compiler_params=_SC_PARAMS,
    scratch_types=_EDGE_SCRATCH + [
        pltpu.VMEM((NPT, D_HID), _f32),
        pltpu.VMEM((NPT, D_HID), _f32),
        pltpu.VMEM((NPT, D_HID), _f32),
        pltpu.VMEM((NPT, D_HID), _f32),
        pltpu.VMEM((NPT, D_HID), _f32),
        pltpu.VMEM((NPT, D_HID), _f32),
        pltpu.VMEM((16,), _f32),
        pltpu.VMEM((D_OUT, 16), _f32),
        pltpu.VMEM((D_OUT, NPT), _f32),
    ],
)
def _layer2_kernel(s1_hbm, g1_hbm, dis_hbm, b1_hbm, w2t_hbm, edges_hbm,
                   zero_hbm, out_hbm,
                   sidx, didx, rows, tbl, acc, gsem, ssem, stsem,
                   p0b, p1b, g1b, db, g2b, accb, b1b, w2b, lb):
    c = lax.axis_index("c")
    s = lax.axis_index("s")
    w = s * 2 + c

    pltpu.sync_copy(edges_hbm.at[0, w], sidx)
    pltpu.sync_copy(edges_hbm.at[1, w], didx)
    pltpu.sync_copy(zero_hbm.at[s], acc.at[pl.ds(s * NPT, NPT)])

    pltpu.make_async_copy(s1_hbm.at[0, s], p0b, stsem).start()
    pltpu.make_async_copy(s1_hbm.at[1, s], p1b, stsem).start()
    pltpu.make_async_copy(g1_hbm.at[s], g1b, stsem).start()
    pltpu.make_async_copy(dis_hbm.at[s], db, stsem).start()
    pltpu.make_async_copy(b1_hbm, b1b, stsem).start()
    pltpu.make_async_copy(w2t_hbm, w2b, stsem).start()
    pltpu.make_async_copy(s1_hbm.at[0, s], p0b, stsem).wait()
    pltpu.make_async_copy(s1_hbm.at[1, s], p1b, stsem).wait()
    pltpu.make_async_copy(g1_hbm.at[s], g1b, stsem).wait()
    pltpu.make_async_copy(dis_hbm.at[s], db, stsem).wait()
    pltpu.make_async_copy(b1_hbm, b1b, stsem).wait()
    pltpu.make_async_copy(w2t_hbm, w2b, stsem).wait()

    b1v = b1b[...]

    def grow(rr, carry):
        d = db[rr, :]
        a = d * (p0b[rr, :] + p1b[rr, :] + g1b[rr, :]) + b1v
        g2b[rr, :] = d * jnp.maximum(a, 0.0)
        return carry

    lax.fori_loop(0, NPT, grow, 0)

    pltpu.sync_copy(g2b, tbl.at[pl.ds(s * NPT, NPT)])
    plsc.subcore_barrier()

    _edge_pipeline(tbl, acc, sidx, didx, rows, gsem, ssem)

    plsc.subcore_barrier()

    pltpu.sync_copy(acc.at[pl.ds(s * NPT, NPT)], accb)
    w20 = w2b[0, :]
    w21 = w2b[1, :]
    gsel = jnp.where(c == 0, 1.0, 0.0).astype(_f32)
    iota16 = lax.iota(jnp.int32, 16)
    zeros16i = jnp.zeros((16,), jnp.int32)

    def lrow(r, carry):
        nidx = r * 16 + iota16
        dcol = plsc.load_gather(db, [nidx, zeros16i])
        l0 = jnp.zeros((16,), _f32)
        l1 = jnp.zeros((16,), _f32)
        for f in range(D_HID):
            fvec = zeros16i + f
            acol = plsc.load_gather(accb, [nidx, fvec])
            gcol = plsc.load_gather(g2b, [nidx, fvec])
            zf = dcol * (acol + gcol * gsel)
            l0 = l0 + zf * w20[f]
            l1 = l1 + zf * w21[f]
        lb[0, pl.ds(r * 16, 16)] = l0
        lb[1, pl.ds(r * 16, 16)] = l1
        return carry

    lax.fori_loop(0, DRT, lrow, 0)

    pltpu.sync_copy(lb, out_hbm.at[c, :, pl.ds(s * NPT, NPT)])


def _h1_body(x_ref, w_ref, o_ref):
    h = jnp.dot(x_ref[...], w_ref[...], preferred_element_type=_f32)
    hp = jnp.concatenate([h, jnp.zeros((N_PAD - N_NODES, D_HID), _f32)], 0)
    o_ref[...] = hp.reshape(16, NPT, D_HID)


def _f_body(l_ref, b_ref, o_ref):
    lg = l_ref[0] + l_ref[1] + b_ref[...]
    m = jnp.max(lg, axis=0, keepdims=True)
    sh = lg - m
    lsm = sh - jnp.log(jnp.sum(jnp.exp(sh), axis=0, keepdims=True))
    o_ref[...] = lsm.T[:N_NODES, :]


def kernel(x, edge_index, W1, b1, W2, b2):
    edges3 = edge_index.reshape(2, NW, CPW, CHUNK)
    zeros = jnp.zeros((16, NPT, D_HID), _f32)

    h1 = pl.pallas_call(
        _h1_body,
        out_shape=jax.ShapeDtypeStruct((16, NPT, D_HID), _f32),
    )(x, W1)

    deg_parts = _deg_kernel(edges3)

    s1, g1, dis = _layer1_kernel(h1, deg_parts, edges3, zeros)

    lparts = _layer2_kernel(s1, g1, dis, b1, W2.T, edges3, zeros)

    out = pl.pallas_call(
        _f_body,
        out_shape=jax.ShapeDtypeStruct((N_NODES, D_OUT), _f32),
    )(lparts, b2.reshape(D_OUT, 1))
    return out
